# baseline (speedup 1.0000x reference)
import functools

import jax
import jax.numpy as jnp
from jax import lax
from jax.experimental import pallas as pl
from jax.experimental.pallas import tpu as pltpu
from jax.experimental.pallas import tpu_sc as plsc

_L = 16


@functools.lru_cache(maxsize=None)
def _make_kernel(N, D, E):
    NC, NS = 2, 16
    NW = NC * NS
    CHUNK = 40
    EP = E // NW
    NCHUNK = EP // CHUNK

    NG = CHUNK // _L

    mesh = plsc.VectorSubcoreMesh(core_axis_name="c", subcore_axis_name="s")

    @functools.partial(
        pl.kernel,
        mesh=mesh,
        compiler_params=pltpu.CompilerParams(needs_layout_passes=False,
                                             use_tc_tiling_on_sc=False),
        out_type=jax.ShapeDtypeStruct((E,), jnp.float32),
        scratch_types=[
            pltpu.VMEM((EP,), jnp.int32),
            pltpu.VMEM((EP,), jnp.int32),
            pltpu.VMEM((CHUNK, D // 2), jnp.int32),
            pltpu.VMEM((CHUNK, D // 2), jnp.int32),
            pltpu.VMEM((CHUNK, D // 2), jnp.int32),
            pltpu.VMEM((CHUNK, D // 2), jnp.int32),
            pltpu.VMEM((EP + _L,), jnp.float32),
            pltpu.VMEM_SHARED((N, D // 2), jnp.int32),
            pltpu.SemaphoreType.DMA,
            pltpu.SemaphoreType.DMA,
        ],
    )
    def k(z_hbm, src_hbm, dst_hbm, out_hbm, sidx, didx,
          srows_a, drows_a, srows_b, drows_b, oall, z_spm, sem_a, sem_b):
        sub = lax.axis_index("s")
        wid = sub * NC + lax.axis_index("c")
        base = wid * EP

        ZR = N // NS
        zsl = pl.ds(sub * ZR, ZR)
        pltpu.sync_copy(z_hbm.at[zsl], z_spm.at[zsl])

        pltpu.sync_copy(src_hbm.at[pl.ds(base, EP)], sidx)
        pltpu.sync_copy(dst_hbm.at[pl.ds(base, EP)], didx)
        plsc.subcore_barrier()

        lane = lax.iota(jnp.int32, _L)
        last = lane == (_L - 1)

        def fire(c, srows, drows, sem):
            sl = pl.ds(c * CHUNK, CHUNK)
            pltpu.async_copy(z_spm.at[sidx.at[sl]], srows, sem)
            pltpu.async_copy(z_spm.at[didx.at[sl]], drows, sem)

        def drain(srows, drows, sem):
            sl = pl.ds(0, CHUNK)
            pltpu.make_async_copy(z_spm.at[sidx.at[sl]], srows, sem).wait()
            pltpu.make_async_copy(z_spm.at[didx.at[sl]], drows, sem).wait()

        def compute(c, srows, drows):
            @plsc.parallel_loop(0, CHUNK, unroll=8)
            def _(e):
                p = []
                for j in range(D // (2 * _L)):
                    sv = plsc.bitcast(srows[e, pl.ds(j * _L, _L)],
                                      jnp.bfloat16)
                    dv = plsc.bitcast(drows[e, pl.ds(j * _L, _L)],
                                      jnp.bfloat16)
                    pa, pb = plsc.unpack(sv * dv,
                                         format=plsc.PackFormat.INTERLEAVED)
                    p.append(pa)
                    p.append(pb)
                while len(p) > 1:
                    p = [p[i] + p[i + 1] for i in range(0, len(p), 2)]
                s = plsc.cumsum(p[0])
                sig = 1.0 / (1.0 + jnp.exp(-s))
                plsc.store_compressed(oall.at[pl.ds(c * CHUNK + e, _L)],
                                      sig, mask=last)

        fire(0, srows_a, drows_a, sem_a)

        def body(kk, carry):
            c = 2 * kk
            fire(c + 1, srows_b, drows_b, sem_b)
            drain(srows_a, drows_a, sem_a)
            compute(c, srows_a, drows_a)

            @pl.when(c + 2 < NCHUNK)
            def _():
                fire(c + 2, srows_a, drows_a, sem_a)

            drain(srows_b, drows_b, sem_b)
            compute(c + 1, srows_b, drows_b)
            return carry

        lax.fori_loop(0, (NCHUNK - 1) // 2, body, 0)
        drain(srows_a, drows_a, sem_a)
        compute(NCHUNK - 1, srows_a, drows_a)

        pltpu.sync_copy(oall.at[pl.ds(0, EP)], out_hbm.at[pl.ds(base, EP)])

    return k


def kernel(z, edge_index):
    N, D = z.shape
    E = edge_index.shape[1]
    ei = edge_index.astype(jnp.int32)
    k = _make_kernel(N, D, E)
    z_pack = jax.lax.bitcast_convert_type(
        z.astype(jnp.bfloat16).reshape(N, D // 2, 2), jnp.int32)
    return k(z_pack, ei[0], ei[1])


# speedup vs baseline: 1.0629x; 1.0629x over previous
import functools

import jax
import jax.numpy as jnp
from jax import lax
from jax.experimental import pallas as pl
from jax.experimental.pallas import tpu as pltpu
from jax.experimental.pallas import tpu_sc as plsc

_L = 16


@functools.lru_cache(maxsize=None)
def _make_kernel(N, D, E):
    NC, NS = 2, 16
    NW = NC * NS
    CHUNK = 128
    EP = E // NW
    MAIN = EP // CHUNK
    TAIL = EP - MAIN * CHUNK
    assert EP * NW == E and MAIN % 2 == 0 and TAIL % 8 == 0
    assert N % NS == 0 and D % (2 * _L) == 0

    mesh = plsc.VectorSubcoreMesh(core_axis_name="c", subcore_axis_name="s")

    @functools.partial(
        pl.kernel,
        mesh=mesh,
        compiler_params=pltpu.CompilerParams(needs_layout_passes=False,
                                             use_tc_tiling_on_sc=False),
        out_type=jax.ShapeDtypeStruct((E,), jnp.float32),
        scratch_types=[
            pltpu.VMEM((EP,), jnp.int32),
            pltpu.VMEM((EP,), jnp.int32),
            pltpu.VMEM((CHUNK, D // 2), jnp.int32),
            pltpu.VMEM((CHUNK, D // 2), jnp.int32),
            pltpu.VMEM((CHUNK, D // 2), jnp.int32),
            pltpu.VMEM((CHUNK, D // 2), jnp.int32),
            pltpu.VMEM((EP + _L,), jnp.float32),
            pltpu.VMEM_SHARED((N, D // 2), jnp.int32),
            pltpu.SemaphoreType.DMA,
            pltpu.SemaphoreType.DMA,
        ],
    )
    def k(z_hbm, src_hbm, dst_hbm, out_hbm, sidx, didx,
          srows_a, drows_a, srows_b, drows_b, oall, z_spm, sem_a, sem_b):
        sub = lax.axis_index("s")
        wid = sub * NC + lax.axis_index("c")
        base = wid * EP

        ZR = N // NS
        zsl = pl.ds(sub * ZR, ZR)
        pltpu.sync_copy(z_hbm.at[zsl], z_spm.at[zsl])

        pltpu.sync_copy(src_hbm.at[pl.ds(base, EP)], sidx)
        pltpu.sync_copy(dst_hbm.at[pl.ds(base, EP)], didx)
        plsc.subcore_barrier()

        lane = lax.iota(jnp.int32, _L)
        last = lane == (_L - 1)

        def fire(off, n, srows, drows, sem):
            sl = pl.ds(off, n)
            pltpu.async_copy(z_spm.at[sidx.at[sl]],
                             srows.at[pl.ds(0, n)], sem)
            pltpu.async_copy(z_spm.at[didx.at[sl]],
                             drows.at[pl.ds(0, n)], sem)

        def drain(n, srows, drows, sem):
            sl = pl.ds(0, n)
            pltpu.make_async_copy(z_spm.at[sidx.at[sl]],
                                  srows.at[sl], sem).wait()
            pltpu.make_async_copy(z_spm.at[didx.at[sl]],
                                  drows.at[sl], sem).wait()

        def compute(off, n, srows, drows):
            @plsc.parallel_loop(0, n, unroll=8)
            def _(e):
                p = []
                for j in range(D // (2 * _L)):
                    sv = plsc.bitcast(srows[e, pl.ds(j * _L, _L)],
                                      jnp.bfloat16)
                    dv = plsc.bitcast(drows[e, pl.ds(j * _L, _L)],
                                      jnp.bfloat16)
                    pa, pb = plsc.unpack(sv * dv,
                                         format=plsc.PackFormat.INTERLEAVED)
                    p.append(pa)
                    p.append(pb)
                while len(p) > 1:
                    p = [p[i] + p[i + 1] for i in range(0, len(p), 2)]
                s = plsc.cumsum(p[0])
                sig = 1.0 / (1.0 + jnp.exp(-s))
                plsc.store_compressed(oall.at[pl.ds(off + e, _L)],
                                      sig, mask=last)

        fire(0, CHUNK, srows_a, drows_a, sem_a)

        def body(kk, carry):
            c = 2 * kk
            fire((c + 1) * CHUNK, CHUNK, srows_b, drows_b, sem_b)
            drain(CHUNK, srows_a, drows_a, sem_a)
            compute(c * CHUNK, CHUNK, srows_a, drows_a)

            @pl.when(c + 2 < MAIN)
            def _():
                fire((c + 2) * CHUNK, CHUNK, srows_a, drows_a, sem_a)

            drain(CHUNK, srows_b, drows_b, sem_b)
            compute((c + 1) * CHUNK, CHUNK, srows_b, drows_b)
            return carry

        lax.fori_loop(0, MAIN // 2, body, 0)

        if TAIL:
            fire(MAIN * CHUNK, TAIL, srows_a, drows_a, sem_a)
            drain(TAIL, srows_a, drows_a, sem_a)
            compute(MAIN * CHUNK, TAIL, srows_a, drows_a)

        pltpu.sync_copy(oall.at[pl.ds(0, EP)], out_hbm.at[pl.ds(base, EP)])

    return k


def kernel(z, edge_index):
    N, D = z.shape
    E = edge_index.shape[1]
    ei = edge_index.astype(jnp.int32)
    k = _make_kernel(N, D, E)
    z_pack = jax.lax.bitcast_convert_type(
        z.astype(jnp.bfloat16).reshape(N, D // 2, 2), jnp.int32)
    return k(z_pack, ei[0], ei[1])


# speedup vs baseline: 1.0679x; 1.0047x over previous
import functools

import jax
import jax.numpy as jnp
from jax import lax
from jax.experimental import pallas as pl
from jax.experimental.pallas import tpu as pltpu
from jax.experimental.pallas import tpu_sc as plsc

_L = 16


@functools.lru_cache(maxsize=None)
def _make_kernel(N, D, E):
    NC, NS = 2, 16
    NW = NC * NS
    CHUNK = 128
    EP = E // NW
    MAIN = EP // CHUNK
    TAIL = EP - MAIN * CHUNK
    assert EP * NW == E and MAIN % 2 == 0 and TAIL % 8 == 0
    assert N % NS == 0 and D % (2 * _L) == 0

    mesh = plsc.VectorSubcoreMesh(core_axis_name="c", subcore_axis_name="s")

    @functools.partial(
        pl.kernel,
        mesh=mesh,
        compiler_params=pltpu.CompilerParams(needs_layout_passes=False,
                                             use_tc_tiling_on_sc=False),
        out_type=jax.ShapeDtypeStruct((E,), jnp.float32),
        scratch_types=[
            pltpu.VMEM((EP,), jnp.int32),
            pltpu.VMEM((EP,), jnp.int32),
            pltpu.VMEM((CHUNK, D // 2), jnp.int32),
            pltpu.VMEM((CHUNK, D // 2), jnp.int32),
            pltpu.VMEM((CHUNK, D // 2), jnp.int32),
            pltpu.VMEM((CHUNK, D // 2), jnp.int32),
            pltpu.VMEM((EP + _L,), jnp.float32),
            pltpu.VMEM_SHARED((N, D // 2), jnp.int32),
            pltpu.SemaphoreType.DMA,
            pltpu.SemaphoreType.DMA,
            pltpu.SemaphoreType.DMA,
            pltpu.SemaphoreType.DMA,
        ],
    )
    def k(z_hbm, src_hbm, dst_hbm, out_hbm, sidx, didx,
          srows_a, drows_a, srows_b, drows_b, oall, z_spm, sem_a, sem_b, sem_a2, sem_b2):
        sub = lax.axis_index("s")
        wid = sub * NC + lax.axis_index("c")
        base = wid * EP

        ZR = N // NS
        zsl = pl.ds(sub * ZR, ZR)
        pltpu.sync_copy(z_hbm.at[zsl], z_spm.at[zsl])

        pltpu.sync_copy(src_hbm.at[pl.ds(base, EP)], sidx)
        pltpu.sync_copy(dst_hbm.at[pl.ds(base, EP)], didx)
        plsc.subcore_barrier()

        lane = lax.iota(jnp.int32, _L)
        last = lane == (_L - 1)

        def fire(off, n, srows, drows, sem, sem2):
            sl = pl.ds(off, n)
            pltpu.async_copy(z_spm.at[sidx.at[sl]],
                             srows.at[pl.ds(0, n)], sem)
            pltpu.async_copy(z_spm.at[didx.at[sl]],
                             drows.at[pl.ds(0, n)], sem2)

        def drain(n, srows, drows, sem, sem2):
            sl = pl.ds(0, n)
            pltpu.make_async_copy(z_spm.at[sidx.at[sl]],
                                  srows.at[sl], sem).wait()
            pltpu.make_async_copy(z_spm.at[didx.at[sl]],
                                  drows.at[sl], sem2).wait()

        def compute(off, n, srows, drows):
            @plsc.parallel_loop(0, n, unroll=8)
            def _(e):
                p = []
                for j in range(D // (2 * _L)):
                    sv = plsc.bitcast(srows[e, pl.ds(j * _L, _L)],
                                      jnp.bfloat16)
                    dv = plsc.bitcast(drows[e, pl.ds(j * _L, _L)],
                                      jnp.bfloat16)
                    pa, pb = plsc.unpack(sv * dv,
                                         format=plsc.PackFormat.INTERLEAVED)
                    p.append(pa)
                    p.append(pb)
                while len(p) > 1:
                    p = [p[i] + p[i + 1] for i in range(0, len(p), 2)]
                s = plsc.cumsum(p[0])
                sig = 1.0 / (1.0 + jnp.exp(-s))
                plsc.store_compressed(oall.at[pl.ds(off + e, _L)],
                                      sig, mask=last)

        fire(0, CHUNK, srows_a, drows_a, sem_a, sem_a2)

        def body(kk, carry):
            c = 2 * kk
            fire((c + 1) * CHUNK, CHUNK, srows_b, drows_b, sem_b, sem_b2)
            drain(CHUNK, srows_a, drows_a, sem_a, sem_a2)
            compute(c * CHUNK, CHUNK, srows_a, drows_a)

            @pl.when(c + 2 < MAIN)
            def _():
                fire((c + 2) * CHUNK, CHUNK, srows_a, drows_a, sem_a, sem_a2)

            drain(CHUNK, srows_b, drows_b, sem_b, sem_b2)
            compute((c + 1) * CHUNK, CHUNK, srows_b, drows_b)
            return carry

        lax.fori_loop(0, MAIN // 2, body, 0)

        if TAIL:
            fire(MAIN * CHUNK, TAIL, srows_a, drows_a, sem_a, sem_a2)
            drain(TAIL, srows_a, drows_a, sem_a, sem_a2)
            compute(MAIN * CHUNK, TAIL, srows_a, drows_a)

        pltpu.sync_copy(oall.at[pl.ds(0, EP)], out_hbm.at[pl.ds(base, EP)])

    return k


def kernel(z, edge_index):
    N, D = z.shape
    E = edge_index.shape[1]
    ei = edge_index.astype(jnp.int32)
    k = _make_kernel(N, D, E)
    z_pack = jax.lax.bitcast_convert_type(
        z.astype(jnp.bfloat16).reshape(N, D // 2, 2), jnp.int32)
    return k(z_pack, ei[0], ei[1])
